# baseline (speedup 1.0000x reference)
import functools

import jax
import jax.numpy as jnp
from jax import lax
from jax.experimental import pallas as pl
from jax.experimental.pallas import tpu as pltpu
from jax.experimental.pallas import tpu_sc as plsc

_NUM_EMB = 1024

_info = plsc.get_sparse_core_info()
_NC, _NS, _L = _info.num_cores, _info.num_subcores, _info.num_lanes
_NW = _NC * _NS


def _vq_body(x_ref, e2_ref, esq_ref, idx_ref, loss_ref,
             loss_acc, iota_scr, *, n_steps, n_total):
    b = pl.program_id(0)

    xb = x_ref[0]
    e2 = e2_ref[...]

    @pl.when(b == 0)
    def _mk_iota():
        iota_scr[...] = lax.broadcasted_iota(
            jnp.int32, iota_scr.shape, 0).astype(jnp.float32)

    xsq = jnp.sum(xb * xb, axis=0, keepdims=True)
    scores = lax.dot_general(e2, xb, (((0,), (0,)), ((), ())),
                             preferred_element_type=jnp.float32)
    dist = (xsq + scores) + esq_ref[...]

    m = jnp.min(dist, axis=0, keepdims=True)
    idx_f = jnp.min(jnp.where(dist == m, iota_scr[...], 2048.0),
                    axis=0, keepdims=True)
    idx_ref[0] = idx_f.astype(jnp.int32)

    part = jnp.sum(m)

    @pl.when(b == 0)
    def _init():
        loss_acc[0, 0] = part

    @pl.when(b > 0)
    def _acc():
        loss_acc[0, 0] += part

    @pl.when(b == n_steps - 1)
    def _fin():
        loss_ref[0, 0] = loss_acc[0, 0] / n_total


def _tc_distance_argmin(x3, em2, esq):
    B, C, hw = x3.shape
    body = functools.partial(_vq_body, n_steps=B,
                             n_total=float(B * hw * C))
    return pl.pallas_call(
        body,
        grid=(B,),
        in_specs=[
            pl.BlockSpec((1, C, hw), lambda b: (b, 0, 0)),
            pl.BlockSpec((C, _NUM_EMB), lambda b: (0, 0)),
            pl.BlockSpec((_NUM_EMB, 1), lambda b: (0, 0)),
        ],
        out_specs=[
            pl.BlockSpec((1, 1, hw), lambda b: (b, 0, 0)),
            pl.BlockSpec((1, 1), lambda b: (0, 0),
                         memory_space=pltpu.MemorySpace.SMEM),
        ],
        out_shape=[
            jax.ShapeDtypeStruct((B, 1, hw), jnp.int32),
            jax.ShapeDtypeStruct((1, 1), jnp.float32),
        ],
        scratch_shapes=[
            pltpu.SMEM((1, 1), jnp.float32),
            pltpu.VMEM((_NUM_EMB, 1024), jnp.float32),
        ],
    )(x3, em2, esq)


def _sc_gather_bincount(table_t, idx_flat, n_tok):
    b_per_w = n_tok // _NW
    mesh = plsc.VectorSubcoreMesh(core_axis_name="c", subcore_axis_name="s")

    @functools.partial(
        pl.kernel, mesh=mesh,
        compiler_params=pltpu.CompilerParams(use_tc_tiling_on_sc=False),
        out_type=[
            jax.ShapeDtypeStruct((n_tok, table_t.shape[1]), jnp.float32),
            jax.ShapeDtypeStruct((_NC, _NUM_EMB), jnp.float32),
        ],
        scratch_types=[
            pltpu.VMEM((b_per_w,), jnp.int32),
            pltpu.VMEM((b_per_w, table_t.shape[1]), jnp.float32),
            pltpu.VMEM((b_per_w,), jnp.float32),
            pltpu.VMEM((_NUM_EMB,), jnp.float32),
            pltpu.VMEM_SHARED((_NUM_EMB,), jnp.float32),
            pltpu.SemaphoreType.DMA,
        ],
    )
    def k(table_hbm, idx_hbm, out_hbm, cnt_hbm,
          idx_v, rows_v, ones_v, zer_v, shared_cnt, sem):
        cid = lax.axis_index("c")
        sid = lax.axis_index("s")
        wid = sid * _NC + cid
        base = wid * b_per_w
        pltpu.sync_copy(idx_hbm.at[pl.ds(base, b_per_w)], idx_v)
        pltpu.async_copy(table_hbm.at[idx_v], rows_v, sem).wait()
        pltpu.sync_copy(rows_v, out_hbm.at[pl.ds(base, b_per_w)])

        ones = jnp.ones((_L,), jnp.float32)
        zers = jnp.zeros((_L,), jnp.float32)
        for j in range(b_per_w // _L):
            ones_v[pl.ds(j * _L, _L)] = ones
        for j in range(_NUM_EMB // _L):
            zer_v[pl.ds(j * _L, _L)] = zers

        @pl.when(sid == 0)
        def _zero():
            pltpu.sync_copy(zer_v, shared_cnt)

        plsc.subcore_barrier()
        pltpu.sync_copy(ones_v, shared_cnt.at[idx_v], add=True)
        plsc.subcore_barrier()

        @pl.when(sid == 0)
        def _cout():
            pltpu.sync_copy(shared_cnt, cnt_hbm.at[cid])

    return k(table_t, idx_flat)


def _ent_body(cnt_ref, ent_ref, *, n_tokens):
    c = cnt_ref[...]
    tot = jnp.sum(c, axis=0, keepdims=True)
    probs = tot / jnp.float32(n_tokens)
    ent_ref[0, 0] = -jnp.sum(probs * jnp.log(probs + 1e-10))


def _tc_entropy(cnt, n_tokens):
    return pl.pallas_call(
        functools.partial(_ent_body, n_tokens=n_tokens),
        out_specs=pl.BlockSpec(memory_space=pltpu.MemorySpace.SMEM),
        out_shape=jax.ShapeDtypeStruct((1, 1), jnp.float32),
    )(cnt)


def kernel(x, embedding_table):
    B, C, H, W = x.shape
    hw = H * W
    n_tokens = B * hw

    x3 = x.reshape(B, C, hw)
    em2 = -2.0 * embedding_table
    esq = jnp.sum(embedding_table ** 2, axis=0).reshape(_NUM_EMB, 1)
    table_t = embedding_table.T

    idx, loss = _tc_distance_argmin(x3, em2, esq)
    idx_flat = idx.reshape(n_tokens)

    rows, cnt = _sc_gather_bincount(table_t, idx_flat, n_tokens)
    ent = _tc_entropy(cnt, n_tokens)

    quantized = jnp.transpose(
        rows.reshape(B, hw, C), (0, 2, 1)).reshape(B, C, H, W)
    loss_s = loss[0, 0]
    return (quantized, loss_s, loss_s, ent[0, 0], idx.reshape(B, hw))


# speedup vs baseline: 1.0358x; 1.0358x over previous
import functools

import jax
import jax.numpy as jnp
from jax import lax
from jax.experimental import pallas as pl
from jax.experimental.pallas import tpu as pltpu
from jax.experimental.pallas import tpu_sc as plsc

_NUM_EMB = 1024

_info = plsc.get_sparse_core_info()
_NC, _NS, _L = _info.num_cores, _info.num_subcores, _info.num_lanes
_NW = _NC * _NS


def _vq_body(x_ref, e2_ref, idx_ref, loss_ref,
             loss_acc, iota_scr, *, n_steps, n_total):
    b = pl.program_id(0)

    x_t = x_ref[0]
    e2 = e2_ref[...]

    @pl.when(b == 0)
    def _mk_iota():
        iota_scr[...] = lax.broadcasted_iota(
            jnp.int32, iota_scr.shape, 1).astype(jnp.float32)

    xsq = jnp.sum(x_t * x_t, axis=1, keepdims=True)
    esq = 0.25 * jnp.sum(e2 * e2, axis=0, keepdims=True)
    scores = jnp.dot(x_t, e2, preferred_element_type=jnp.float32)
    dist = (xsq + scores) + esq

    m = jnp.min(dist, axis=1, keepdims=True)
    idx_f = jnp.min(jnp.where(dist == m, iota_scr[...], 2048.0),
                    axis=1, keepdims=True)
    idx_ref[0] = idx_f.astype(jnp.int32)

    part = jnp.sum(m)

    @pl.when(b == 0)
    def _init():
        loss_acc[0, 0] = part

    @pl.when(b > 0)
    def _acc():
        loss_acc[0, 0] += part

    @pl.when(b == n_steps - 1)
    def _fin():
        loss_ref[0, 0] = loss_acc[0, 0] / n_total


def _tc_distance_argmin(x_t, em2):
    B, hw, C = x_t.shape
    body = functools.partial(_vq_body, n_steps=B,
                             n_total=float(B * hw * C))
    return pl.pallas_call(
        body,
        grid=(B,),
        in_specs=[
            pl.BlockSpec((1, hw, C), lambda b: (b, 0, 0)),
            pl.BlockSpec((C, _NUM_EMB), lambda b: (0, 0)),
        ],
        out_specs=[
            pl.BlockSpec((1, hw, 1), lambda b: (b, 0, 0)),
            pl.BlockSpec((1, 1), lambda b: (0, 0),
                         memory_space=pltpu.MemorySpace.SMEM),
        ],
        out_shape=[
            jax.ShapeDtypeStruct((B, hw, 1), jnp.int32),
            jax.ShapeDtypeStruct((1, 1), jnp.float32),
        ],
        scratch_shapes=[
            pltpu.SMEM((1, 1), jnp.float32),
            pltpu.VMEM((1024, _NUM_EMB), jnp.float32),
        ],
    )(x_t, em2)


def _sc_gather_bincount(table_t, idx_flat, n_tok):
    b_per_w = n_tok // _NW
    mesh = plsc.VectorSubcoreMesh(core_axis_name="c", subcore_axis_name="s")

    @functools.partial(
        pl.kernel, mesh=mesh,
        compiler_params=pltpu.CompilerParams(use_tc_tiling_on_sc=False),
        out_type=[
            jax.ShapeDtypeStruct((n_tok, table_t.shape[1]), jnp.float32),
            jax.ShapeDtypeStruct((_NC, _NUM_EMB), jnp.float32),
        ],
        scratch_types=[
            pltpu.VMEM((b_per_w,), jnp.int32),
            pltpu.VMEM((b_per_w, table_t.shape[1]), jnp.float32),
            pltpu.VMEM((b_per_w,), jnp.float32),
            pltpu.VMEM((_NUM_EMB,), jnp.float32),
            pltpu.VMEM_SHARED((_NUM_EMB,), jnp.float32),
            pltpu.SemaphoreType.DMA,
        ],
    )
    def k(table_hbm, idx_hbm, out_hbm, cnt_hbm,
          idx_v, rows_v, ones_v, zer_v, shared_cnt, sem):
        cid = lax.axis_index("c")
        sid = lax.axis_index("s")
        wid = sid * _NC + cid
        base = wid * b_per_w
        pltpu.sync_copy(idx_hbm.at[pl.ds(base, b_per_w)], idx_v)
        pltpu.async_copy(table_hbm.at[idx_v], rows_v, sem).wait()
        pltpu.sync_copy(rows_v, out_hbm.at[pl.ds(base, b_per_w)])

        ones = jnp.ones((_L,), jnp.float32)
        zers = jnp.zeros((_L,), jnp.float32)
        for j in range(b_per_w // _L):
            ones_v[pl.ds(j * _L, _L)] = ones
        for j in range(_NUM_EMB // _L):
            zer_v[pl.ds(j * _L, _L)] = zers

        @pl.when(sid == 0)
        def _zero():
            pltpu.sync_copy(zer_v, shared_cnt)

        plsc.subcore_barrier()
        pltpu.sync_copy(ones_v, shared_cnt.at[idx_v], add=True)
        plsc.subcore_barrier()

        @pl.when(sid == 0)
        def _cout():
            pltpu.sync_copy(shared_cnt, cnt_hbm.at[cid])

    return k(table_t, idx_flat)


def _ent_body(cnt_ref, ent_ref, *, n_tokens):
    c = cnt_ref[...]
    tot = jnp.sum(c, axis=0, keepdims=True)
    probs = tot / jnp.float32(n_tokens)
    ent_ref[0, 0] = -jnp.sum(probs * jnp.log(probs + 1e-10))


def _tc_entropy(cnt, n_tokens):
    return pl.pallas_call(
        functools.partial(_ent_body, n_tokens=n_tokens),
        out_specs=pl.BlockSpec(memory_space=pltpu.MemorySpace.SMEM),
        out_shape=jax.ShapeDtypeStruct((1, 1), jnp.float32),
    )(cnt)


def kernel(x, embedding_table):
    B, C, H, W = x.shape
    hw = H * W
    n_tokens = B * hw

    x_t = jnp.transpose(x.reshape(B, C, hw), (0, 2, 1))
    em2 = -2.0 * embedding_table
    table_t = embedding_table.T

    idx, loss = _tc_distance_argmin(x_t, em2)
    idx_flat = idx.reshape(n_tokens)

    rows, cnt = _sc_gather_bincount(table_t, idx_flat, n_tokens)
    ent = _tc_entropy(cnt, n_tokens)

    quantized = jnp.transpose(
        rows.reshape(B, hw, C), (0, 2, 1)).reshape(B, C, H, W)
    loss_s = loss[0, 0]
    return (quantized, loss_s, loss_s, ent[0, 0], idx.reshape(B, hw))
